# baseline (speedup 1.0000x reference)
import functools

import jax
import jax.numpy as jnp
from jax import lax
from jax.experimental import pallas as pl
from jax.experimental.pallas import tpu as pltpu
from jax.experimental.pallas import tpu_sc as plsc

_VOCAB = 50257
_MAX_POS = 2048
_EMBED_DIM = 1024

_NC = 2
_NS = 16
_NW = _NC * _NS

_PPW = _MAX_POS // _NW
_CH = 16
_NH = _PPW // _CH
_NGS = 5


def _make_emb_kernel(n_batch):
    nch = _NH * n_batch

    mesh = plsc.VectorSubcoreMesh(core_axis_name="c", subcore_axis_name="s")

    @functools.partial(
        pl.kernel,
        mesh=mesh,
        out_type=jax.ShapeDtypeStruct((n_batch * _MAX_POS, _EMBED_DIM),
                                      jnp.float32),
        scratch_types=[
            pltpu.VMEM((n_batch, _PPW), jnp.int32),
            pltpu.VMEM((_NGS, _CH, _EMBED_DIM), jnp.float32),
            pltpu.VMEM((2, _CH, _EMBED_DIM), jnp.float32),
            pltpu.SemaphoreType.DMA((_NGS,)),
            pltpu.SemaphoreType.DMA((2,)),
            pltpu.SemaphoreType.DMA((_NGS,)),
        ],
    )
    def emb(idx_hbm, wte_hbm, wpe_hbm, out_hbm, idx_v, gbuf, pbuf,
            gsem, psem, osem):
        wid = lax.axis_index("s") * _NC + lax.axis_index("c")
        pos0 = wid * _PPW
        pltpu.async_copy(wpe_hbm.at[pl.ds(pos0, _CH)], pbuf.at[0],
                         psem.at[0])
        i_desc = [
            pltpu.async_copy(idx_hbm.at[b, pl.ds(pos0, _PPW)], idx_v.at[b],
                             psem.at[1])
            for b in range(n_batch)
        ]
        for d in i_desc:
            d.wait()

        def start_gather(c):
            h, b = divmod(c, n_batch)
            s = c % _NGS
            return pltpu.async_copy(
                wte_hbm.at[idx_v.at[b, pl.ds(h * _CH, _CH)]],
                gbuf.at[s], gsem.at[s])

        def start_pload(h):
            return pltpu.async_copy(
                wpe_hbm.at[pl.ds(pos0 + h * _CH, _CH)],
                pbuf.at[h % 2], psem.at[h % 2])

        def start_gather_t(c):
            h = lax.div(c, n_batch)
            b = lax.rem(c, n_batch)
            s = lax.rem(c, _NGS)
            return pltpu.async_copy(
                wte_hbm.at[idx_v.at[b, pl.ds(h * _CH, _CH)]],
                gbuf.at[s], gsem.at[s])

        def out_copy_t(c, start):
            h = lax.div(c, n_batch)
            b = lax.rem(c, n_batch)
            s = lax.rem(c, _NGS)
            row = b * _MAX_POS + pos0 + h * _CH
            d = pltpu.make_async_copy(
                gbuf.at[s], out_hbm.at[pl.ds(row, _CH)], osem.at[s])
            if start:
                d.start()
            return d

        for c in range(3):
            start_gather(c)

        @pl.loop(0, nch)
        def chunk(c):
            h = lax.div(c, n_batch)
            b = lax.rem(c, n_batch)
            s = lax.rem(c, _NGS)
            hp = lax.rem(h, 2)

            @pl.when(b == 0)
            def _():
                pltpu.make_async_copy(
                    wpe_hbm.at[pl.ds(pos0 + h * _CH, _CH)],
                    pbuf.at[hp], psem.at[hp]).wait()

                @pl.when(h + 1 < _NH)
                def _():
                    pltpu.async_copy(
                        wpe_hbm.at[pl.ds(pos0 + (h + 1) * _CH, _CH)],
                        pbuf.at[1 - hp], psem.at[1 - hp])

            pltpu.make_async_copy(
                wte_hbm.at[idx_v.at[b, pl.ds(h * _CH, _CH)]],
                gbuf.at[s], gsem.at[s]).wait()

            @pl.when(c >= 2)
            def _():
                out_copy_t(c - 2, start=False).wait()

            @pl.when(c + 3 < nch)
            def _():
                start_gather_t(c + 3)

            @plsc.parallel_loop(0, _CH * _EMBED_DIM // 16, unroll=8)
            def add_vec(v):
                r = lax.shift_right_logical(v, 6)
                col = pl.multiple_of(
                    lax.shift_left(lax.bitwise_and(v, 63), 4), 16)
                sl = pl.ds(col, 16)
                plsc.addupdate(gbuf.at[s, r, sl], pbuf[hp, r, sl])

            out_copy_t(c, start=True)

        for c in range(nch - 2, nch):
            h, b = divmod(c, n_batch)
            s = c % _NGS
            row = b * _MAX_POS + pos0 + h * _CH
            pltpu.make_async_copy(
                gbuf.at[s], out_hbm.at[pl.ds(row, _CH)], osem.at[s]).wait()

    return emb


def kernel(input_ids, wte, wpe):
    input_shape = input_ids.shape
    seq = input_shape[-1]
    ids = input_ids.reshape(-1, seq).astype(jnp.int32)
    n_batch = ids.shape[0]
    out = _make_emb_kernel(n_batch)(ids, wte, wpe)
    return out.reshape(n_batch, seq, _EMBED_DIM)


# speedup vs baseline: 1.0118x; 1.0118x over previous
import functools

import jax
import jax.numpy as jnp
from jax import lax
from jax.experimental import pallas as pl
from jax.experimental.pallas import tpu as pltpu
from jax.experimental.pallas import tpu_sc as plsc

_VOCAB = 50257
_MAX_POS = 2048
_EMBED_DIM = 1024

_NC = 2
_NS = 16
_NW = _NC * _NS

_PPW = _MAX_POS // _NW
_CH = 16
_NH = _PPW // _CH
_NGS = 4


def _make_emb_kernel(n_batch):
    nch = _NH * n_batch

    mesh = plsc.VectorSubcoreMesh(core_axis_name="c", subcore_axis_name="s")

    @functools.partial(
        pl.kernel,
        mesh=mesh,
        out_type=jax.ShapeDtypeStruct((n_batch * _MAX_POS, _EMBED_DIM),
                                      jnp.float32),
        scratch_types=[
            pltpu.VMEM((n_batch, _PPW), jnp.int32),
            pltpu.VMEM((_NGS, _CH, _EMBED_DIM), jnp.float32),
            pltpu.VMEM((2, _CH, _EMBED_DIM), jnp.float32),
            pltpu.SemaphoreType.DMA((_NGS,)),
            pltpu.SemaphoreType.DMA((2,)),
            pltpu.SemaphoreType.DMA((_NGS,)),
        ],
    )
    def emb(idx_hbm, wte_hbm, wpe_hbm, out_hbm, idx_v, gbuf, pbuf,
            gsem, psem, osem):
        wid = lax.axis_index("s") * _NC + lax.axis_index("c")
        pos0 = wid * _PPW
        pltpu.async_copy(wpe_hbm.at[pl.ds(pos0, _CH)], pbuf.at[0],
                         psem.at[0])
        i_desc = [
            pltpu.async_copy(idx_hbm.at[b, pl.ds(pos0, _PPW)], idx_v.at[b],
                             psem.at[1])
            for b in range(n_batch)
        ]
        for d in i_desc:
            d.wait()

        def start_gather(c):
            h, b = divmod(c, n_batch)
            s = c % _NGS
            return pltpu.async_copy(
                wte_hbm.at[idx_v.at[b, pl.ds(h * _CH, _CH)]],
                gbuf.at[s], gsem.at[s])

        def start_pload(h):
            return pltpu.async_copy(
                wpe_hbm.at[pl.ds(pos0 + h * _CH, _CH)],
                pbuf.at[h % 2], psem.at[h % 2])

        def start_gather_t(c):
            h = lax.div(c, n_batch)
            b = lax.rem(c, n_batch)
            s = lax.rem(c, _NGS)
            return pltpu.async_copy(
                wte_hbm.at[idx_v.at[b, pl.ds(h * _CH, _CH)]],
                gbuf.at[s], gsem.at[s])

        def out_copy_t(c, start):
            h = lax.div(c, n_batch)
            b = lax.rem(c, n_batch)
            s = lax.rem(c, _NGS)
            row = b * _MAX_POS + pos0 + h * _CH
            d = pltpu.make_async_copy(
                gbuf.at[s], out_hbm.at[pl.ds(row, _CH)], osem.at[s])
            if start:
                d.start()
            return d

        for c in range(2):
            start_gather(c)

        @pl.loop(0, nch)
        def chunk(c):
            h = lax.div(c, n_batch)
            b = lax.rem(c, n_batch)
            s = lax.rem(c, _NGS)
            hp = lax.rem(h, 2)

            @pl.when(b == 0)
            def _():
                pltpu.make_async_copy(
                    wpe_hbm.at[pl.ds(pos0 + h * _CH, _CH)],
                    pbuf.at[hp], psem.at[hp]).wait()

                @pl.when(h + 1 < _NH)
                def _():
                    pltpu.async_copy(
                        wpe_hbm.at[pl.ds(pos0 + (h + 1) * _CH, _CH)],
                        pbuf.at[1 - hp], psem.at[1 - hp])

            pltpu.make_async_copy(
                wte_hbm.at[idx_v.at[b, pl.ds(h * _CH, _CH)]],
                gbuf.at[s], gsem.at[s]).wait()

            @pl.when(c >= 2)
            def _():
                out_copy_t(c - 2, start=False).wait()

            @pl.when(c + 2 < nch)
            def _():
                start_gather_t(c + 2)

            @plsc.parallel_loop(0, _CH * _EMBED_DIM // 16, unroll=8)
            def add_vec(v):
                r = lax.shift_right_logical(v, 6)
                col = pl.multiple_of(
                    lax.shift_left(lax.bitwise_and(v, 63), 4), 16)
                sl = pl.ds(col, 16)
                plsc.addupdate(gbuf.at[s, r, sl], pbuf[hp, r, sl])

            out_copy_t(c, start=True)

        for c in range(nch - 2, nch):
            h, b = divmod(c, n_batch)
            s = c % _NGS
            row = b * _MAX_POS + pos0 + h * _CH
            pltpu.make_async_copy(
                gbuf.at[s], out_hbm.at[pl.ds(row, _CH)], osem.at[s]).wait()

    return emb


def kernel(input_ids, wte, wpe):
    input_shape = input_ids.shape
    seq = input_shape[-1]
    ids = input_ids.reshape(-1, seq).astype(jnp.int32)
    n_batch = ids.shape[0]
    out = _make_emb_kernel(n_batch)(ids, wte, wpe)
    return out.reshape(n_batch, seq, _EMBED_DIM)


# speedup vs baseline: 1.0138x; 1.0020x over previous
import functools

import jax
import jax.numpy as jnp
from jax import lax
from jax.experimental import pallas as pl
from jax.experimental.pallas import tpu as pltpu
from jax.experimental.pallas import tpu_sc as plsc

_VOCAB = 50257
_MAX_POS = 2048
_EMBED_DIM = 1024

_NC = 2
_NS = 16
_NW = _NC * _NS

_PPW = _MAX_POS // _NW
_CH = 16
_NH = _PPW // _CH
_NGS = 4


def _make_emb_kernel(n_batch):
    nch = _NH * n_batch

    mesh = plsc.VectorSubcoreMesh(core_axis_name="c", subcore_axis_name="s")

    @functools.partial(
        pl.kernel,
        mesh=mesh,
        out_type=jax.ShapeDtypeStruct((n_batch * _MAX_POS, _EMBED_DIM),
                                      jnp.float32),
        scratch_types=[
            pltpu.VMEM((n_batch, _PPW), jnp.int32),
            pltpu.VMEM((_NGS, _CH, _EMBED_DIM), jnp.float32),
            pltpu.VMEM((2, _CH, _EMBED_DIM), jnp.float32),
            pltpu.SemaphoreType.DMA((_NGS,)),
            pltpu.SemaphoreType.DMA((2,)),
            pltpu.SemaphoreType.DMA((_NGS,)),
        ],
    )
    def emb(idx_hbm, wte_hbm, wpe_hbm, out_hbm, idx_v, gbuf, pbuf,
            gsem, psem, osem):
        wid = lax.axis_index("s") * _NC + lax.axis_index("c")
        pos0 = wid * _PPW
        pltpu.async_copy(wpe_hbm.at[pl.ds(pos0, _CH)], pbuf.at[0],
                         psem.at[0])
        i_desc = [
            pltpu.async_copy(idx_hbm.at[b, pl.ds(pos0, _PPW)], idx_v.at[b],
                             psem.at[1])
            for b in range(n_batch)
        ]
        for d in i_desc:
            d.wait()

        def start_gather(c):
            h, b = divmod(c, n_batch)
            s = c % _NGS
            return pltpu.async_copy(
                wte_hbm.at[idx_v.at[b, pl.ds(h * _CH, _CH)]],
                gbuf.at[s], gsem.at[s])

        def start_gather_t(c):
            h = lax.div(c, n_batch)
            b = lax.rem(c, n_batch)
            s = lax.rem(c, _NGS)
            return pltpu.async_copy(
                wte_hbm.at[idx_v.at[b, pl.ds(h * _CH, _CH)]],
                gbuf.at[s], gsem.at[s])

        def out_copy_t(c, start):
            h = lax.div(c, n_batch)
            b = lax.rem(c, n_batch)
            s = lax.rem(c, _NGS)
            row = b * _MAX_POS + pos0 + h * _CH
            d = pltpu.make_async_copy(
                gbuf.at[s], out_hbm.at[pl.ds(row, _CH)], osem.at[s])
            if start:
                d.start()
            return d

        for c in range(2):
            start_gather(c)

        @pl.loop(0, nch)
        def chunk(c):
            h = lax.div(c, n_batch)
            b = lax.rem(c, n_batch)
            s = lax.rem(c, _NGS)
            hp = lax.rem(h, 2)

            @pl.when(b == 0)
            def _():
                pltpu.make_async_copy(
                    wpe_hbm.at[pl.ds(pos0 + h * _CH, _CH)],
                    pbuf.at[hp], psem.at[hp]).wait()

                @pl.when(h + 1 < _NH)
                def _():
                    pltpu.async_copy(
                        wpe_hbm.at[pl.ds(pos0 + (h + 1) * _CH, _CH)],
                        pbuf.at[1 - hp], psem.at[1 - hp])

            pltpu.make_async_copy(
                wte_hbm.at[idx_v.at[b, pl.ds(h * _CH, _CH)]],
                gbuf.at[s], gsem.at[s]).wait()

            @pl.when(c >= 2)
            def _():
                out_copy_t(c - 2, start=False).wait()

            @pl.when(c + 2 < nch)
            def _():
                start_gather_t(c + 2)

            @plsc.parallel_loop(0, _CH * _EMBED_DIM // 16, unroll=8)
            def add_vec(v):
                r = lax.shift_right_logical(v, 6)
                col = pl.multiple_of(
                    lax.shift_left(lax.bitwise_and(v, 63), 4), 16)
                sl = pl.ds(col, 16)
                plsc.addupdate(gbuf.at[s, r, sl], pbuf[hp, r, sl])

            out_copy_t(c, start=True)

        for c in range(nch - 2, nch):
            h, b = divmod(c, n_batch)
            s = c % _NGS
            row = b * _MAX_POS + pos0 + h * _CH
            pltpu.make_async_copy(
                gbuf.at[s], out_hbm.at[pl.ds(row, _CH)], osem.at[s]).wait()

    return emb


def kernel(input_ids, wte, wpe):
    input_shape = input_ids.shape
    seq = input_shape[-1]
    ids = input_ids.reshape(-1, seq).astype(jnp.int32)
    n_batch = ids.shape[0]
    out = _make_emb_kernel(n_batch)(ids, wte, wpe)
    return out.reshape(n_batch, seq, _EMBED_DIM)
